# Initial kernel scaffold; baseline (speedup 1.0000x reference)
import jax
import jax.numpy as jnp
from jax.experimental import pallas as pl


def kernel(x_scalar, x_vector, edge_index, edge_attr, pos, W1, b1, W2, b2, w00_0, w11_0, w01_1, w10_1, w11_1, centers, widths):
    raise NotImplementedError("write your pallas kernel here")


# speedup vs baseline: 5.2998x; 5.2998x over previous
import functools
import math

import jax
import jax.numpy as jnp
from jax import lax
from jax.experimental import pallas as pl
from jax.experimental.pallas import tpu as pltpu

N = 10000
E = 160000
C = 256
NUM_RBF = 16
CUTOFF = 10.0

ALPHA1 = 1.0 / math.sqrt(3.0 * C)
SH0C = 1.0 / (2.0 * math.sqrt(math.pi))
SH1C = math.sqrt(3.0 / (4.0 * math.pi))

TN = 400
TE = 800

SRC_W = 2048
DST_W = 272
POS_W = 8
MSG_W = 1024


def _k1_body(xs_ref, xvT_ref, pos_ref, wsc_ref, wv_ref, src_ref, dst_ref, posp_ref):
    xs = xs_ref[...]
    sc = jnp.dot(xs, wsc_ref[...], preferred_element_type=jnp.float32)
    vparts = []
    for m in range(3):
        vm = jnp.dot(xvT_ref[m], wv_ref[...], preferred_element_type=jnp.float32)
        vparts.append(vm)
    src_ref[...] = jnp.concatenate(
        [sc[:, :512], vparts[0][:, :256], vparts[1][:, :256], vparts[2][:, :256],
         vparts[0][:, 256:], vparts[1][:, 256:], vparts[2][:, 256:]], axis=1)
    pos = pos_ref[...]
    pad_d = jnp.zeros((pos.shape[0], DST_W - C - 8), dtype=jnp.float32)
    dst_ref[...] = jnp.concatenate([sc[:, 512:768], pos, pad_d], axis=1)
    posp_ref[...] = pos


def _node_tables(xs, xvT, pos8, Wsc, Wv):
    grid = (N // TN,)
    return pl.pallas_call(
        _k1_body,
        grid=grid,
        in_specs=[
            pl.BlockSpec((TN, C), lambda i: (i, 0)),
            pl.BlockSpec((3, TN, C), lambda i: (0, i, 0)),
            pl.BlockSpec((TN, POS_W), lambda i: (i, 0)),
            pl.BlockSpec((C, 768), lambda i: (0, 0)),
            pl.BlockSpec((C, 512), lambda i: (0, 0)),
        ],
        out_specs=[
            pl.BlockSpec((TN, SRC_W), lambda i: (i, 0)),
            pl.BlockSpec((TN, DST_W), lambda i: (i, 0)),
            pl.BlockSpec((TN, POS_W), lambda i: (i, 0)),
        ],
        out_shape=[
            jax.ShapeDtypeStruct((N, SRC_W), jnp.float32),
            jax.ShapeDtypeStruct((N, DST_W), jnp.float32),
            jax.ShapeDtypeStruct((N, POS_W), jnp.float32),
        ],
    )(xs, xvT, pos8, Wsc, Wv)


def _k4_body(srcg_ref, dpg_ref, prg_ref, w1r_ref, b1_ref, cen_ref, wid_ref, msg_ref):
    posr = prg_ref[:, :3]
    posc = dpg_ref[:, C:C + 3]
    diff = posr - posc
    dd = jnp.sum(diff * diff, axis=1, keepdims=True)
    dist = jnp.sqrt(dd)
    d = jnp.minimum(dist, CUTOFF)
    z = (d - cen_ref[...]) / wid_ref[...]
    rbf = jnp.exp(-(z * z)) * (1.0 - (d / CUTOFF) ** 2)
    rbfw = jnp.dot(rbf, w1r_ref[...], preferred_element_type=jnp.float32)
    pre = srcg_ref[:, :C] + dpg_ref[:, :C] + rbfw + b1_ref[...]
    h = pre * (1.0 / (1.0 + jnp.exp(-pre)))
    sh1 = SH1C * diff / (dist + 1e-8)
    sP = srcg_ref[:, C:2 * C]
    outs = [h]
    for m in range(3):
        m1, m2 = (m + 1) % 3, (m + 2) % 3
        vPm = srcg_ref[:, 512 + C * m:512 + C * (m + 1)]
        vQ1 = srcg_ref[:, 1280 + C * m1:1280 + C * (m1 + 1)]
        vQ2 = srcg_ref[:, 1280 + C * m2:1280 + C * (m2 + 1)]
        om = (sP * sh1[:, m:m + 1] + vPm
              + vQ1 * sh1[:, m2:m2 + 1] - vQ2 * sh1[:, m1:m1 + 1])
        outs.append(om)
    msg_ref[...] = jnp.concatenate(outs, axis=1)


def _edge_stage(srcg, dpg, prg, W1r, b1r, cen, wid):
    grid = (E // TE,)
    return pl.pallas_call(
        _k4_body,
        grid=grid,
        in_specs=[
            pl.BlockSpec((TE, SRC_W), lambda i: (i, 0)),
            pl.BlockSpec((TE, DST_W), lambda i: (i, 0)),
            pl.BlockSpec((TE, POS_W), lambda i: (i, 0)),
            pl.BlockSpec((NUM_RBF, C), lambda i: (0, 0)),
            pl.BlockSpec((1, C), lambda i: (0, 0)),
            pl.BlockSpec((1, NUM_RBF), lambda i: (0, 0)),
            pl.BlockSpec((1, NUM_RBF), lambda i: (0, 0)),
        ],
        out_specs=pl.BlockSpec((TE, MSG_W), lambda i: (i, 0)),
        out_shape=jax.ShapeDtypeStruct((E, MSG_W), jnp.float32),
    )(srcg, dpg, prg, W1r, b1r, cen, wid)


def _k6_body(m_ref, xs_ref, xvT_ref, w2_ref, b2_ref, outs_ref, outv_ref):
    H = m_ref[:, :C]
    so = jnp.dot(H, w2_ref[...], preferred_element_type=jnp.float32) + b2_ref[...]
    so = so * (1.0 / (1.0 + jnp.exp(-so)))
    gates = 1.0 / (1.0 + jnp.exp(-so))
    outs_ref[...] = xs_ref[...] + so
    for m in range(3):
        outv_ref[m] = xvT_ref[m] + m_ref[:, C * (m + 1):C * (m + 2)] * gates


def _finish(msum, xs, xvT, W2, b2r):
    grid = (N // TN,)
    return pl.pallas_call(
        _k6_body,
        grid=grid,
        in_specs=[
            pl.BlockSpec((TN, MSG_W), lambda i: (i, 0)),
            pl.BlockSpec((TN, C), lambda i: (i, 0)),
            pl.BlockSpec((3, TN, C), lambda i: (0, i, 0)),
            pl.BlockSpec((C, C), lambda i: (0, 0)),
            pl.BlockSpec((1, C), lambda i: (0, 0)),
        ],
        out_specs=[
            pl.BlockSpec((TN, C), lambda i: (i, 0)),
            pl.BlockSpec((3, TN, C), lambda i: (0, i, 0)),
        ],
        out_shape=[
            jax.ShapeDtypeStruct((N, C), jnp.float32),
            jax.ShapeDtypeStruct((3, N, C), jnp.float32),
        ],
    )(msum, xs, xvT, W2, b2r)


@jax.jit
def kernel(x_scalar, x_vector, edge_index, edge_attr, pos, W1, b1, W2, b2,
           w00_0, w11_0, w01_1, w10_1, w11_1, centers, widths):
    del edge_attr, w00_0, w11_0
    xs = x_scalar
    xvT = jnp.transpose(x_vector, (2, 0, 1))
    pos8 = jnp.pad(pos, ((0, 0), (0, POS_W - 3)))
    row = edge_index[0]
    col = edge_index[1]

    Wsc = jnp.concatenate([W1[:C], ALPHA1 * w01_1, W1[C:2 * C]], axis=1)
    Wv = jnp.concatenate([(ALPHA1 * SH0C) * w10_1,
                          (ALPHA1 / math.sqrt(2.0)) * w11_1], axis=1)
    W1r = W1[2 * C:]
    b1r = b1.reshape(1, C)
    b2r = b2.reshape(1, C)
    cen = centers.reshape(1, NUM_RBF)
    wid = widths.reshape(1, NUM_RBF)

    src_tab, dst_tab, pos_tab = _node_tables(xs, xvT, pos8, Wsc, Wv)

    srcg = jnp.take(src_tab, row, axis=0)
    dpg = jnp.take(dst_tab, col, axis=0)
    prg = jnp.take(pos_tab, row, axis=0)

    msgs = _edge_stage(srcg, dpg, prg, W1r, b1r, cen, wid)

    msum = jax.ops.segment_sum(msgs, col, num_segments=N)

    out_s, outvT = _finish(msum, xs, xvT, W2, b2r)
    out_v = jnp.transpose(outvT, (1, 2, 0))
    return (out_s, out_v)
